# Initial kernel scaffold; baseline (speedup 1.0000x reference)
import jax
import jax.numpy as jnp
from jax.experimental import pallas as pl


def kernel(x, batch):
    raise NotImplementedError("write your pallas kernel here")


# speedup vs baseline: 4.6224x; 4.6224x over previous
import functools

import jax
import jax.numpy as jnp
from jax.experimental import pallas as pl
from jax.experimental.pallas import tpu as pltpu

_B = 512
_K = 3
_ROWS = 1024

_NEG_INF = float("-inf")


def _lex_gt(v_a, i_a, v_b, i_b):
    return (v_a > v_b) | ((v_a == v_b) & (i_a < i_b))


def _pass1_kernel(x_ref, seg_ref, mean_ref, ssum_ref, idx_ref,
                  cnt_ref, tv_ref, ti_ref, *, n_rows, n_blocks, b, k):
    i = pl.program_id(0)
    r, d = x_ref.shape

    @pl.when(i == 0)
    def _init():
        ssum_ref[...] = jnp.zeros_like(ssum_ref)
        cnt_ref[...] = jnp.zeros_like(cnt_ref)
        tv_ref[...] = jnp.full_like(tv_ref, _NEG_INF)
        ti_ref[...] = jnp.full_like(ti_ref, n_rows)

    gidx = i * r + jax.lax.broadcasted_iota(jnp.int32, (r, 1), 0)
    row_ok = gidx < n_rows
    x = jnp.where(row_ok, x_ref[...], 0.0)
    seg = seg_ref[...]
    lanes = jax.lax.broadcasted_iota(jnp.int32, (1, b), 1)
    oh_bool = (seg == lanes) & row_ok
    oh = oh_bool.astype(jnp.float32)

    ssum_ref[...] += jax.lax.dot_general(
        oh, x, (((0,), (0,)), ((), ())),
        preferred_element_type=jnp.float32)
    cnt_ref[...] += jax.lax.dot_general(
        oh, jnp.ones((r, 1), jnp.float32), (((0,), (0,)), ((), ())),
        preferred_element_type=jnp.float32)

    vals = x[:, d - 1:d]
    alive = oh_bool
    for _ in range(k):
        mv = jnp.where(alive, vals, _NEG_INF)
        m = jnp.max(mv, axis=0, keepdims=True)
        cand = jnp.where(alive & (mv == m), gidx, n_rows)
        ix = jnp.min(cand, axis=0, keepdims=True)
        cv, ci = m, ix
        for row in range(k):
            tv_r = tv_ref[row:row + 1, :]
            ti_r = ti_ref[row:row + 1, :]
            takes = _lex_gt(cv, ci, tv_r, ti_r)
            tv_ref[row:row + 1, :] = jnp.where(takes, cv, tv_r)
            ti_ref[row:row + 1, :] = jnp.where(takes, ci, ti_r)
            cv = jnp.where(takes, tv_r, cv)
            ci = jnp.where(takes, ti_r, ci)
        alive = alive & (gidx != ix)

    @pl.when(i == n_blocks - 1)
    def _fin():
        mean_ref[...] = ssum_ref[...] / jnp.maximum(cnt_ref[...], 1.0)
        idx_ref[...] = ti_ref[...]


def _pass2_kernel(idx_ref, x_ref, out_ref, *, n_rows, b, k):
    i = pl.program_id(0)
    r, d = x_ref.shape

    @pl.when(i == 0)
    def _init():
        out_ref[...] = jnp.zeros_like(out_ref)

    gidx = i * r + jax.lax.broadcasted_iota(jnp.int32, (r, 1), 0)
    row_ok = gidx < n_rows
    x = jnp.where(row_ok, x_ref[...], 0.0)
    for kk in range(k):
        tik = idx_ref[kk:kk + 1, :]
        sel = (gidx == tik).astype(jnp.float32)
        out_ref[:, kk * d:(kk + 1) * d] += jax.lax.dot_general(
            sel, x, (((0,), (0,)), ((), ())),
            preferred_element_type=jnp.float32)


@jax.jit
def kernel(x, batch):
    n, d = x.shape
    b, k, r = _B, _K, _ROWS
    nb = pl.cdiv(n, r)
    seg = batch.astype(jnp.int32).reshape(n, 1)

    mean, ssum, idx8 = pl.pallas_call(
        functools.partial(_pass1_kernel, n_rows=n, n_blocks=nb, b=b, k=k),
        grid=(nb,),
        in_specs=[
            pl.BlockSpec((r, d), lambda i: (i, 0)),
            pl.BlockSpec((r, 1), lambda i: (i, 0)),
        ],
        out_specs=[
            pl.BlockSpec((b, d), lambda i: (0, 0)),
            pl.BlockSpec((b, d), lambda i: (0, 0)),
            pl.BlockSpec((8, b), lambda i: (0, 0)),
        ],
        out_shape=[
            jax.ShapeDtypeStruct((b, d), jnp.float32),
            jax.ShapeDtypeStruct((b, d), jnp.float32),
            jax.ShapeDtypeStruct((8, b), jnp.int32),
        ],
        scratch_shapes=[
            pltpu.VMEM((b, 1), jnp.float32),
            pltpu.VMEM((8, b), jnp.float32),
            pltpu.VMEM((8, b), jnp.int32),
        ],
    )(x, seg)

    topk = pl.pallas_call(
        functools.partial(_pass2_kernel, n_rows=n, b=b, k=k),
        grid=(nb,),
        in_specs=[
            pl.BlockSpec((8, b), lambda i: (0, 0)),
            pl.BlockSpec((r, d), lambda i: (i, 0)),
        ],
        out_specs=pl.BlockSpec((b, k * d), lambda i: (0, 0)),
        out_shape=jax.ShapeDtypeStruct((b, k * d), jnp.float32),
    )(idx8, x)

    return jnp.concatenate([mean, ssum, topk], axis=-1)


# speedup vs baseline: 5.2540x; 1.1366x over previous
import functools

import jax
import jax.numpy as jnp
from jax.experimental import pallas as pl
from jax.experimental.pallas import tpu as pltpu

_B = 512
_K = 3
_ROWS = 800

_NEG_INF = float("-inf")


def _lex_gt(v_a, i_a, v_b, i_b):
    return (v_a > v_b) | ((v_a == v_b) & (i_a < i_b))


def _pass1_kernel(x_ref, seg_ref, mean_ref, ssum_ref, idx_ref,
                  cnt_ref, tv_ref, ti_ref, *, n_rows, n_blocks, b, k):
    i = pl.program_id(0)
    r, d = x_ref.shape

    @pl.when(i == 0)
    def _init():
        ssum_ref[...] = jnp.zeros_like(ssum_ref)
        cnt_ref[...] = jnp.zeros_like(cnt_ref)
        tv_ref[...] = jnp.full_like(tv_ref, _NEG_INF)
        ti_ref[...] = jnp.full_like(ti_ref, n_rows)

    gidx = i * r + jax.lax.broadcasted_iota(jnp.int32, (r, 1), 0)
    x = x_ref[...]
    seg = seg_ref[...]
    lanes = jax.lax.broadcasted_iota(jnp.int32, (1, b), 1)
    oh_bool = seg == lanes
    oh = oh_bool.astype(jnp.float32)

    ssum_ref[...] += jax.lax.dot_general(
        oh, x, (((0,), (0,)), ((), ())),
        preferred_element_type=jnp.float32)
    cnt_ref[...] += jax.lax.dot_general(
        oh, jnp.ones((r, 1), jnp.float32), (((0,), (0,)), ((), ())),
        preferred_element_type=jnp.float32)

    vals = x[:, d - 1:d]
    mv = jnp.where(oh_bool, vals, _NEG_INF)
    sentinel = jnp.int32(n_rows)
    for _ in range(k):
        m = jnp.max(mv, axis=0, keepdims=True)
        cand = jnp.where(mv == m, gidx, n_rows)
        ix = jnp.min(cand, axis=0, keepdims=True)
        ix = jnp.where(m == _NEG_INF, sentinel, ix)
        cv, ci = m, ix
        for row in range(k):
            tv_r = tv_ref[row:row + 1, :]
            ti_r = ti_ref[row:row + 1, :]
            takes = _lex_gt(cv, ci, tv_r, ti_r)
            tv_ref[row:row + 1, :] = jnp.where(takes, cv, tv_r)
            ti_ref[row:row + 1, :] = jnp.where(takes, ci, ti_r)
            cv = jnp.where(takes, tv_r, cv)
            ci = jnp.where(takes, ti_r, ci)
        mv = jnp.where(gidx == ix, _NEG_INF, mv)

    @pl.when(i == n_blocks - 1)
    def _fin():
        mean_ref[...] = ssum_ref[...] / jnp.maximum(cnt_ref[...], 1.0)
        idx_ref[...] = ti_ref[...]


def _pass2_kernel(idx_ref, x_ref, out_ref, *, n_rows, b, k):
    i = pl.program_id(0)
    r, d = x_ref.shape

    @pl.when(i == 0)
    def _init():
        out_ref[...] = jnp.zeros_like(out_ref)

    gidx = i * r + jax.lax.broadcasted_iota(jnp.int32, (r, 1), 0)
    x = x_ref[...]
    for kk in range(k):
        tik = idx_ref[kk:kk + 1, :]
        sel = (gidx == tik).astype(jnp.float32)
        out_ref[:, kk * d:(kk + 1) * d] += jax.lax.dot_general(
            sel, x, (((0,), (0,)), ((), ())),
            preferred_element_type=jnp.float32)


@jax.jit
def kernel(x, batch):
    n, d = x.shape
    b, k, r = _B, _K, _ROWS
    seg = batch.astype(jnp.int32).reshape(n, 1)
    n_pad = -(-n // r) * r
    if n_pad != n:
        x = jnp.pad(x, ((0, n_pad - n), (0, 0)))
        seg = jnp.pad(seg, ((0, n_pad - n), (0, 0)), constant_values=b)
    nb = n_pad // r

    mean, ssum, idx8 = pl.pallas_call(
        functools.partial(_pass1_kernel, n_rows=n, n_blocks=nb, b=b, k=k),
        grid=(nb,),
        in_specs=[
            pl.BlockSpec((r, d), lambda i: (i, 0)),
            pl.BlockSpec((r, 1), lambda i: (i, 0)),
        ],
        out_specs=[
            pl.BlockSpec((b, d), lambda i: (0, 0)),
            pl.BlockSpec((b, d), lambda i: (0, 0)),
            pl.BlockSpec((8, b), lambda i: (0, 0)),
        ],
        out_shape=[
            jax.ShapeDtypeStruct((b, d), jnp.float32),
            jax.ShapeDtypeStruct((b, d), jnp.float32),
            jax.ShapeDtypeStruct((8, b), jnp.int32),
        ],
        scratch_shapes=[
            pltpu.VMEM((b, 1), jnp.float32),
            pltpu.VMEM((8, b), jnp.float32),
            pltpu.VMEM((8, b), jnp.int32),
        ],
    )(x, seg)

    topk = pl.pallas_call(
        functools.partial(_pass2_kernel, n_rows=n, b=b, k=k),
        grid=(nb,),
        in_specs=[
            pl.BlockSpec((8, b), lambda i: (0, 0)),
            pl.BlockSpec((r, d), lambda i: (i, 0)),
        ],
        out_specs=pl.BlockSpec((b, k * d), lambda i: (0, 0)),
        out_shape=jax.ShapeDtypeStruct((b, k * d), jnp.float32),
    )(idx8, x)

    return jnp.concatenate([mean, ssum, topk], axis=-1)


# speedup vs baseline: 8.2659x; 1.5733x over previous
import functools

import jax
import jax.numpy as jnp
from jax.experimental import pallas as pl
from jax.experimental.pallas import tpu as pltpu
from jax.experimental.pallas import tpu_sc as plsc

_B = 512
_K = 3
_ROWS = 800

_NEG_INF = float("-inf")


def _lex_gt(v_a, i_a, v_b, i_b):
    return (v_a > v_b) | ((v_a == v_b) & (i_a < i_b))


def _pass1_kernel(x_ref, seg_ref, mean_ref, ssum_ref, idx_ref, cnt_out_ref,
                  cnt_ref, tv_ref, ti_ref, *, n_rows, n_blocks, b, k):
    i = pl.program_id(0)
    r, d = x_ref.shape

    @pl.when(i == 0)
    def _init():
        ssum_ref[...] = jnp.zeros_like(ssum_ref)
        cnt_ref[...] = jnp.zeros_like(cnt_ref)
        tv_ref[...] = jnp.full_like(tv_ref, _NEG_INF)
        ti_ref[...] = jnp.full_like(ti_ref, n_rows)

    gidx = i * r + jax.lax.broadcasted_iota(jnp.int32, (r, 1), 0)
    x = x_ref[...]
    seg = seg_ref[...]
    lanes = jax.lax.broadcasted_iota(jnp.int32, (1, b), 1)
    oh_bool = seg == lanes
    oh = oh_bool.astype(jnp.float32)

    ssum_ref[...] += jax.lax.dot_general(
        oh, x, (((0,), (0,)), ((), ())),
        preferred_element_type=jnp.float32)
    cnt_ref[...] += jax.lax.dot_general(
        oh, jnp.ones((r, 1), jnp.float32), (((0,), (0,)), ((), ())),
        preferred_element_type=jnp.float32)

    vals = x[:, d - 1:d]
    mv = jnp.where(oh_bool, vals, _NEG_INF)
    sentinel = jnp.int32(n_rows)
    for _ in range(k):
        m = jnp.max(mv, axis=0, keepdims=True)
        cand = jnp.where(mv == m, gidx, n_rows)
        ix = jnp.min(cand, axis=0, keepdims=True)
        ix = jnp.where(m == _NEG_INF, sentinel, ix)
        cv, ci = m, ix
        for row in range(k):
            tv_r = tv_ref[row:row + 1, :]
            ti_r = ti_ref[row:row + 1, :]
            takes = _lex_gt(cv, ci, tv_r, ti_r)
            tv_ref[row:row + 1, :] = jnp.where(takes, cv, tv_r)
            ti_ref[row:row + 1, :] = jnp.where(takes, ci, ti_r)
            cv = jnp.where(takes, tv_r, cv)
            ci = jnp.where(takes, ti_r, ci)
        mv = jnp.where(gidx == ix, _NEG_INF, mv)

    @pl.when(i == n_blocks - 1)
    def _fin():
        mean_ref[...] = ssum_ref[...] / jnp.maximum(cnt_ref[...], 1.0)
        idx_ref[...] = ti_ref[...]
        cnt_out_ref[...] = cnt_ref[...]


def _sc_gather(x, idx_flat, m_rows, d):
    info = plsc.get_sparse_core_info()
    nw = info.num_cores * info.num_subcores
    rows_per_w = m_rows // nw
    mesh = plsc.VectorSubcoreMesh(core_axis_name="c", subcore_axis_name="s")

    @functools.partial(
        pl.kernel, mesh=mesh,
        out_type=jax.ShapeDtypeStruct((m_rows, d), jnp.float32),
        scratch_types=[
            pltpu.VMEM((rows_per_w,), jnp.int32),
            pltpu.VMEM((rows_per_w, d), jnp.float32),
            pltpu.SemaphoreType.DMA,
        ],
    )
    def _gather(table_hbm, idx_hbm, out_hbm, idx_v, rows_v, sem):
        wid = jax.lax.axis_index("s") * info.num_cores + jax.lax.axis_index("c")
        base = wid * rows_per_w
        pltpu.sync_copy(idx_hbm.at[pl.ds(base, rows_per_w)], idx_v)
        pltpu.async_copy(table_hbm.at[idx_v], rows_v, sem).wait()
        pltpu.sync_copy(rows_v, out_hbm.at[pl.ds(base, rows_per_w)])

    return _gather(x, idx_flat)


def _epilogue_kernel(cnt_ref, g_ref, out_ref, *, b, k, d):
    for kk in range(k):
        valid = cnt_ref[...] > jnp.float32(kk)
        out_ref[:, kk * d:(kk + 1) * d] = jnp.where(
            valid, g_ref[kk * b:(kk + 1) * b, :], 0.0)


@jax.jit
def kernel(x, batch):
    n, d = x.shape
    b, k, r = _B, _K, _ROWS
    seg = batch.astype(jnp.int32).reshape(n, 1)
    n_pad = -(-n // r) * r
    if n_pad != n:
        x = jnp.pad(x, ((0, n_pad - n), (0, 0)))
        seg = jnp.pad(seg, ((0, n_pad - n), (0, 0)), constant_values=b)
    nb = n_pad // r

    mean, ssum, idx8, cnt = pl.pallas_call(
        functools.partial(_pass1_kernel, n_rows=n, n_blocks=nb, b=b, k=k),
        grid=(nb,),
        in_specs=[
            pl.BlockSpec((r, d), lambda i: (i, 0)),
            pl.BlockSpec((r, 1), lambda i: (i, 0)),
        ],
        out_specs=[
            pl.BlockSpec((b, d), lambda i: (0, 0)),
            pl.BlockSpec((b, d), lambda i: (0, 0)),
            pl.BlockSpec((8, b), lambda i: (0, 0)),
            pl.BlockSpec((b, 1), lambda i: (0, 0)),
        ],
        out_shape=[
            jax.ShapeDtypeStruct((b, d), jnp.float32),
            jax.ShapeDtypeStruct((b, d), jnp.float32),
            jax.ShapeDtypeStruct((8, b), jnp.int32),
            jax.ShapeDtypeStruct((b, 1), jnp.float32),
        ],
        scratch_shapes=[
            pltpu.VMEM((b, 1), jnp.float32),
            pltpu.VMEM((8, b), jnp.float32),
            pltpu.VMEM((8, b), jnp.int32),
        ],
    )(x, seg)

    idx_flat = jnp.minimum(idx8.reshape(-1)[:k * b], n - 1)
    g = _sc_gather(x, idx_flat, k * b, d)

    topk = pl.pallas_call(
        functools.partial(_epilogue_kernel, b=b, k=k, d=d),
        grid=(1,),
        in_specs=[
            pl.BlockSpec((b, 1), lambda i: (0, 0)),
            pl.BlockSpec((k * b, d), lambda i: (0, 0)),
        ],
        out_specs=pl.BlockSpec((b, k * d), lambda i: (0, 0)),
        out_shape=jax.ShapeDtypeStruct((b, k * d), jnp.float32),
    )(cnt, g)

    return jnp.concatenate([mean, ssum, topk], axis=-1)


# speedup vs baseline: 9.9619x; 1.2052x over previous
import functools

import jax
import jax.numpy as jnp
from jax.experimental import pallas as pl
from jax.experimental.pallas import tpu as pltpu
from jax.experimental.pallas import tpu_sc as plsc

_B = 512
_K = 3
_ROWS = 2000

_NEG_INF = float("-inf")


def _lex_gt(v_a, i_a, v_b, i_b):
    return (v_a > v_b) | ((v_a == v_b) & (i_a < i_b))


def _pass1_kernel(x_ref, seg_ref, mean_ref, ssum_ref, idx_ref, cnt_out_ref,
                  cnt_ref, tv_ref, ti_ref, *, n_rows, n_blocks, b, k):
    i = pl.program_id(0)
    r, d = x_ref.shape

    @pl.when(i == 0)
    def _init():
        ssum_ref[...] = jnp.zeros_like(ssum_ref)
        cnt_ref[...] = jnp.zeros_like(cnt_ref)
        tv_ref[...] = jnp.full_like(tv_ref, _NEG_INF)
        ti_ref[...] = jnp.full_like(ti_ref, n_rows)

    gidx = i * r + jax.lax.broadcasted_iota(jnp.int32, (r, 1), 0)
    x = x_ref[...]
    seg = seg_ref[...]
    lanes = jax.lax.broadcasted_iota(jnp.int32, (1, b), 1)
    oh_bool = seg == lanes
    oh = oh_bool.astype(jnp.float32)

    ssum_ref[...] += jax.lax.dot_general(
        oh, x, (((0,), (0,)), ((), ())),
        preferred_element_type=jnp.float32)
    cnt_ref[...] += jax.lax.dot_general(
        oh, jnp.ones((r, 1), jnp.float32), (((0,), (0,)), ((), ())),
        preferred_element_type=jnp.float32)

    vals = x[:, d - 1:d]
    mv = jnp.where(oh_bool, vals, _NEG_INF)
    sentinel = jnp.int32(n_rows)
    for kk in range(k):
        m = jnp.max(mv, axis=0, keepdims=True)
        cand = jnp.where(mv == m, gidx, n_rows)
        ix = jnp.min(cand, axis=0, keepdims=True)
        ix = jnp.where(m == _NEG_INF, sentinel, ix)
        cv, ci = m, ix
        for row in range(k):
            tv_r = tv_ref[row:row + 1, :]
            ti_r = ti_ref[row:row + 1, :]
            takes = _lex_gt(cv, ci, tv_r, ti_r)
            tv_ref[row:row + 1, :] = jnp.where(takes, cv, tv_r)
            ti_ref[row:row + 1, :] = jnp.where(takes, ci, ti_r)
            cv = jnp.where(takes, tv_r, cv)
            ci = jnp.where(takes, ti_r, ci)
        if kk < k - 1:
            mv = jnp.where(gidx == ix, _NEG_INF, mv)

    @pl.when(i == n_blocks - 1)
    def _fin():
        mean_ref[...] = ssum_ref[...] / jnp.maximum(cnt_ref[...], 1.0)
        idx_ref[...] = ti_ref[...]
        cnt_out_ref[...] = cnt_ref[...]


def _sc_gather(x, idx_flat, m_rows, d):
    info = plsc.get_sparse_core_info()
    nw = info.num_cores * info.num_subcores
    rows_per_w = m_rows // nw
    mesh = plsc.VectorSubcoreMesh(core_axis_name="c", subcore_axis_name="s")

    @functools.partial(
        pl.kernel, mesh=mesh,
        out_type=jax.ShapeDtypeStruct((m_rows, d), jnp.float32),
        scratch_types=[
            pltpu.VMEM((rows_per_w,), jnp.int32),
            pltpu.VMEM((rows_per_w, d), jnp.float32),
            pltpu.SemaphoreType.DMA,
        ],
    )
    def _gather(table_hbm, idx_hbm, out_hbm, idx_v, rows_v, sem):
        wid = jax.lax.axis_index("s") * info.num_cores + jax.lax.axis_index("c")
        base = wid * rows_per_w
        pltpu.sync_copy(idx_hbm.at[pl.ds(base, rows_per_w)], idx_v)
        pltpu.async_copy(table_hbm.at[idx_v], rows_v, sem).wait()
        pltpu.sync_copy(rows_v, out_hbm.at[pl.ds(base, rows_per_w)])

    return _gather(x, idx_flat)


def _epilogue_kernel(cnt_ref, g_ref, out_ref, *, b, k, d):
    for kk in range(k):
        valid = cnt_ref[...] > jnp.float32(kk)
        out_ref[:, kk * d:(kk + 1) * d] = jnp.where(
            valid, g_ref[kk * b:(kk + 1) * b, :], 0.0)


@jax.jit
def kernel(x, batch):
    n, d = x.shape
    b, k, r = _B, _K, _ROWS
    seg = batch.astype(jnp.int32).reshape(n, 1)
    n_pad = -(-n // r) * r
    if n_pad != n:
        x = jnp.pad(x, ((0, n_pad - n), (0, 0)))
        seg = jnp.pad(seg, ((0, n_pad - n), (0, 0)), constant_values=b)
    nb = n_pad // r

    mean, ssum, idx8, cnt = pl.pallas_call(
        functools.partial(_pass1_kernel, n_rows=n, n_blocks=nb, b=b, k=k),
        grid=(nb,),
        in_specs=[
            pl.BlockSpec((r, d), lambda i: (i, 0)),
            pl.BlockSpec((r, 1), lambda i: (i, 0)),
        ],
        out_specs=[
            pl.BlockSpec((b, d), lambda i: (0, 0)),
            pl.BlockSpec((b, d), lambda i: (0, 0)),
            pl.BlockSpec((8, b), lambda i: (0, 0)),
            pl.BlockSpec((b, 1), lambda i: (0, 0)),
        ],
        out_shape=[
            jax.ShapeDtypeStruct((b, d), jnp.float32),
            jax.ShapeDtypeStruct((b, d), jnp.float32),
            jax.ShapeDtypeStruct((8, b), jnp.int32),
            jax.ShapeDtypeStruct((b, 1), jnp.float32),
        ],
        scratch_shapes=[
            pltpu.VMEM((b, 1), jnp.float32),
            pltpu.VMEM((8, b), jnp.float32),
            pltpu.VMEM((8, b), jnp.int32),
        ],
    )(x, seg)

    idx_flat = jnp.minimum(idx8.reshape(-1)[:k * b], n - 1)
    g = _sc_gather(x, idx_flat, k * b, d)

    topk = pl.pallas_call(
        functools.partial(_epilogue_kernel, b=b, k=k, d=d),
        grid=(1,),
        in_specs=[
            pl.BlockSpec((b, 1), lambda i: (0, 0)),
            pl.BlockSpec((k * b, d), lambda i: (0, 0)),
        ],
        out_specs=pl.BlockSpec((b, k * d), lambda i: (0, 0)),
        out_shape=jax.ShapeDtypeStruct((b, k * d), jnp.float32),
    )(cnt, g)

    return jnp.concatenate([mean, ssum, topk], axis=-1)


# speedup vs baseline: 10.0039x; 1.0042x over previous
import functools

import jax
import jax.numpy as jnp
from jax.experimental import pallas as pl
from jax.experimental.pallas import tpu as pltpu
from jax.experimental.pallas import tpu_sc as plsc

_B = 512
_K = 3
_ROWS = 4000

_NEG_INF = float("-inf")


def _lex_gt(v_a, i_a, v_b, i_b):
    return (v_a > v_b) | ((v_a == v_b) & (i_a < i_b))


def _pass1_kernel(x_ref, seg_ref, mean_ref, ssum_ref, idx_ref, cnt_out_ref,
                  cnt_ref, tv_ref, ti_ref, *, n_rows, n_blocks, b, k):
    i = pl.program_id(0)
    r, d = x_ref.shape

    @pl.when(i == 0)
    def _init():
        ssum_ref[...] = jnp.zeros_like(ssum_ref)
        cnt_ref[...] = jnp.zeros_like(cnt_ref)
        tv_ref[...] = jnp.full_like(tv_ref, _NEG_INF)
        ti_ref[...] = jnp.full_like(ti_ref, n_rows)

    gidx = i * r + jax.lax.broadcasted_iota(jnp.int32, (r, 1), 0)
    x = x_ref[...]
    seg = seg_ref[...]
    lanes = jax.lax.broadcasted_iota(jnp.int32, (1, b), 1)
    oh_bool = seg == lanes
    oh = oh_bool.astype(jnp.float32)

    ssum_ref[...] += jax.lax.dot_general(
        oh, x, (((0,), (0,)), ((), ())),
        preferred_element_type=jnp.float32)
    cnt_ref[...] += jax.lax.dot_general(
        oh, jnp.ones((r, 1), jnp.float32), (((0,), (0,)), ((), ())),
        preferred_element_type=jnp.float32)

    vals = x[:, d - 1:d]
    mv = jnp.where(oh_bool, vals, _NEG_INF)
    sentinel = jnp.int32(n_rows)
    for kk in range(k):
        m = jnp.max(mv, axis=0, keepdims=True)
        cand = jnp.where(mv == m, gidx, n_rows)
        ix = jnp.min(cand, axis=0, keepdims=True)
        ix = jnp.where(m == _NEG_INF, sentinel, ix)
        cv, ci = m, ix
        for row in range(k):
            tv_r = tv_ref[row:row + 1, :]
            ti_r = ti_ref[row:row + 1, :]
            takes = _lex_gt(cv, ci, tv_r, ti_r)
            tv_ref[row:row + 1, :] = jnp.where(takes, cv, tv_r)
            ti_ref[row:row + 1, :] = jnp.where(takes, ci, ti_r)
            cv = jnp.where(takes, tv_r, cv)
            ci = jnp.where(takes, ti_r, ci)
        if kk < k - 1:
            mv = jnp.where(gidx == ix, _NEG_INF, mv)

    @pl.when(i == n_blocks - 1)
    def _fin():
        mean_ref[...] = ssum_ref[...] / jnp.maximum(cnt_ref[...], 1.0)
        idx_ref[...] = ti_ref[...]
        cnt_out_ref[...] = cnt_ref[...]


def _sc_gather(x, idx_flat, m_rows, d):
    info = plsc.get_sparse_core_info()
    nw = info.num_cores * info.num_subcores
    rows_per_w = m_rows // nw
    mesh = plsc.VectorSubcoreMesh(core_axis_name="c", subcore_axis_name="s")

    @functools.partial(
        pl.kernel, mesh=mesh,
        out_type=jax.ShapeDtypeStruct((m_rows, d), jnp.float32),
        scratch_types=[
            pltpu.VMEM((rows_per_w,), jnp.int32),
            pltpu.VMEM((rows_per_w, d), jnp.float32),
            pltpu.SemaphoreType.DMA,
        ],
    )
    def _gather(table_hbm, idx_hbm, out_hbm, idx_v, rows_v, sem):
        wid = jax.lax.axis_index("s") * info.num_cores + jax.lax.axis_index("c")
        base = wid * rows_per_w
        pltpu.sync_copy(idx_hbm.at[pl.ds(base, rows_per_w)], idx_v)
        pltpu.async_copy(table_hbm.at[idx_v], rows_v, sem).wait()
        pltpu.sync_copy(rows_v, out_hbm.at[pl.ds(base, rows_per_w)])

    return _gather(x, idx_flat)


def _epilogue_kernel(cnt_ref, g_ref, out_ref, *, b, k, d):
    for kk in range(k):
        valid = cnt_ref[...] > jnp.float32(kk)
        out_ref[:, kk * d:(kk + 1) * d] = jnp.where(
            valid, g_ref[kk * b:(kk + 1) * b, :], 0.0)


@jax.jit
def kernel(x, batch):
    n, d = x.shape
    b, k, r = _B, _K, _ROWS
    seg = batch.astype(jnp.int32).reshape(n, 1)
    n_pad = -(-n // r) * r
    if n_pad != n:
        x = jnp.pad(x, ((0, n_pad - n), (0, 0)))
        seg = jnp.pad(seg, ((0, n_pad - n), (0, 0)), constant_values=b)
    nb = n_pad // r

    mean, ssum, idx8, cnt = pl.pallas_call(
        functools.partial(_pass1_kernel, n_rows=n, n_blocks=nb, b=b, k=k),
        grid=(nb,),
        in_specs=[
            pl.BlockSpec((r, d), lambda i: (i, 0)),
            pl.BlockSpec((r, 1), lambda i: (i, 0)),
        ],
        out_specs=[
            pl.BlockSpec((b, d), lambda i: (0, 0)),
            pl.BlockSpec((b, d), lambda i: (0, 0)),
            pl.BlockSpec((8, b), lambda i: (0, 0)),
            pl.BlockSpec((b, 1), lambda i: (0, 0)),
        ],
        out_shape=[
            jax.ShapeDtypeStruct((b, d), jnp.float32),
            jax.ShapeDtypeStruct((b, d), jnp.float32),
            jax.ShapeDtypeStruct((8, b), jnp.int32),
            jax.ShapeDtypeStruct((b, 1), jnp.float32),
        ],
        scratch_shapes=[
            pltpu.VMEM((b, 1), jnp.float32),
            pltpu.VMEM((8, b), jnp.float32),
            pltpu.VMEM((8, b), jnp.int32),
        ],
    )(x, seg)

    idx_flat = jnp.minimum(idx8.reshape(-1)[:k * b], n - 1)
    g = _sc_gather(x, idx_flat, k * b, d)

    topk = pl.pallas_call(
        functools.partial(_epilogue_kernel, b=b, k=k, d=d),
        grid=(1,),
        in_specs=[
            pl.BlockSpec((b, 1), lambda i: (0, 0)),
            pl.BlockSpec((k * b, d), lambda i: (0, 0)),
        ],
        out_specs=pl.BlockSpec((b, k * d), lambda i: (0, 0)),
        out_shape=jax.ShapeDtypeStruct((b, k * d), jnp.float32),
    )(cnt, g)

    return jnp.concatenate([mean, ssum, topk], axis=-1)


# speedup vs baseline: 11.6373x; 1.1633x over previous
import functools

import jax
import jax.numpy as jnp
from jax.experimental import pallas as pl
from jax.experimental.pallas import tpu as pltpu
from jax.experimental.pallas import tpu_sc as plsc

_B = 512
_K = 3
_ROWS = 4000

_NEG_INF = float("-inf")


def _lex_gt(v_a, i_a, v_b, i_b):
    return (v_a > v_b) | ((v_a == v_b) & (i_a < i_b))


def _win_extract(seg, vals, gidx, tv_ref, ti_ref, lo, w, n_rows, k):
    lanes_w = lo + jax.lax.broadcasted_iota(jnp.int32, (1, w), 1)
    mvw = jnp.where(seg == lanes_w, vals, _NEG_INF)
    sentinel = jnp.int32(n_rows)
    for kk in range(k):
        m = jnp.max(mvw, axis=0, keepdims=True)
        cand = jnp.where(mvw == m, gidx, n_rows)
        ix = jnp.min(cand, axis=0, keepdims=True)
        ix = jnp.where(m == _NEG_INF, sentinel, ix)
        cv, ci = m, ix
        for row in range(k):
            tv_r = tv_ref[row:row + 1, lo:lo + w]
            ti_r = ti_ref[row:row + 1, lo:lo + w]
            takes = _lex_gt(cv, ci, tv_r, ti_r)
            tv_ref[row:row + 1, lo:lo + w] = jnp.where(takes, cv, tv_r)
            ti_ref[row:row + 1, lo:lo + w] = jnp.where(takes, ci, ti_r)
            cv = jnp.where(takes, tv_r, cv)
            ci = jnp.where(takes, ti_r, ci)
        if kk < k - 1:
            mvw = jnp.where(gidx == ix, _NEG_INF, mvw)


def _pass1_kernel(x_ref, seg_ref, mean_ref, ssum_ref, idx_ref, cnt_out_ref,
                  cnt_ref, tv_ref, ti_ref, *, n_rows, n_blocks, b, k):
    i = pl.program_id(0)
    r, d = x_ref.shape

    @pl.when(i == 0)
    def _init():
        ssum_ref[...] = jnp.zeros_like(ssum_ref)
        cnt_ref[...] = jnp.zeros_like(cnt_ref)
        tv_ref[...] = jnp.full_like(tv_ref, _NEG_INF)
        ti_ref[...] = jnp.full_like(ti_ref, n_rows)

    gidx = i * r + jax.lax.broadcasted_iota(jnp.int32, (r, 1), 0)
    x = x_ref[...]
    seg = seg_ref[...]
    lanes = jax.lax.broadcasted_iota(jnp.int32, (1, b), 1)
    oh_bool = seg == lanes
    oh = oh_bool.astype(jnp.float32)

    ssum_ref[...] += jax.lax.dot_general(
        oh, x, (((0,), (0,)), ((), ())),
        preferred_element_type=jnp.float32)
    cnt_ref[...] += jax.lax.dot_general(
        oh, jnp.ones((r, 1), jnp.float32), (((0,), (0,)), ((), ())),
        preferred_element_type=jnp.float32)

    vals = x[:, d - 1:d]
    s_lo = jnp.min(seg)
    s_hi = jnp.max(seg)
    w = 128 if b % 128 == 0 else b
    for h in range(b // w):
        lo = h * w

        @pl.when(jnp.logical_and(s_lo < lo + w, s_hi >= lo))
        def _window(lo=lo):
            _win_extract(seg, vals, gidx, tv_ref, ti_ref, lo, w, n_rows, k)

    @pl.when(i == n_blocks - 1)
    def _fin():
        mean_ref[...] = ssum_ref[...] / jnp.maximum(cnt_ref[...], 1.0)
        idx_ref[...] = ti_ref[...]
        cnt_out_ref[...] = cnt_ref[...]


def _sc_gather(x, idx_flat, m_rows, d):
    info = plsc.get_sparse_core_info()
    nw = info.num_cores * info.num_subcores
    rows_per_w = m_rows // nw
    mesh = plsc.VectorSubcoreMesh(core_axis_name="c", subcore_axis_name="s")

    @functools.partial(
        pl.kernel, mesh=mesh,
        out_type=jax.ShapeDtypeStruct((m_rows, d), jnp.float32),
        scratch_types=[
            pltpu.VMEM((rows_per_w,), jnp.int32),
            pltpu.VMEM((rows_per_w, d), jnp.float32),
            pltpu.SemaphoreType.DMA,
        ],
    )
    def _gather(table_hbm, idx_hbm, out_hbm, idx_v, rows_v, sem):
        wid = jax.lax.axis_index("s") * info.num_cores + jax.lax.axis_index("c")
        base = wid * rows_per_w
        pltpu.sync_copy(idx_hbm.at[pl.ds(base, rows_per_w)], idx_v)
        pltpu.async_copy(table_hbm.at[idx_v], rows_v, sem).wait()
        pltpu.sync_copy(rows_v, out_hbm.at[pl.ds(base, rows_per_w)])

    return _gather(x, idx_flat)


def _epilogue_kernel(cnt_ref, g_ref, out_ref, *, b, k, d):
    for kk in range(k):
        valid = cnt_ref[...] > jnp.float32(kk)
        out_ref[:, kk * d:(kk + 1) * d] = jnp.where(
            valid, g_ref[kk * b:(kk + 1) * b, :], 0.0)


@jax.jit
def kernel(x, batch):
    n, d = x.shape
    b, k, r = _B, _K, _ROWS
    seg = batch.astype(jnp.int32).reshape(n, 1)
    n_pad = -(-n // r) * r
    if n_pad != n:
        x = jnp.pad(x, ((0, n_pad - n), (0, 0)))
        seg = jnp.pad(seg, ((0, n_pad - n), (0, 0)), constant_values=b)
    nb = n_pad // r

    mean, ssum, idx8, cnt = pl.pallas_call(
        functools.partial(_pass1_kernel, n_rows=n, n_blocks=nb, b=b, k=k),
        grid=(nb,),
        in_specs=[
            pl.BlockSpec((r, d), lambda i: (i, 0)),
            pl.BlockSpec((r, 1), lambda i: (i, 0)),
        ],
        out_specs=[
            pl.BlockSpec((b, d), lambda i: (0, 0)),
            pl.BlockSpec((b, d), lambda i: (0, 0)),
            pl.BlockSpec((8, b), lambda i: (0, 0)),
            pl.BlockSpec((b, 1), lambda i: (0, 0)),
        ],
        out_shape=[
            jax.ShapeDtypeStruct((b, d), jnp.float32),
            jax.ShapeDtypeStruct((b, d), jnp.float32),
            jax.ShapeDtypeStruct((8, b), jnp.int32),
            jax.ShapeDtypeStruct((b, 1), jnp.float32),
        ],
        scratch_shapes=[
            pltpu.VMEM((b, 1), jnp.float32),
            pltpu.VMEM((8, b), jnp.float32),
            pltpu.VMEM((8, b), jnp.int32),
        ],
    )(x, seg)

    idx_flat = jnp.minimum(idx8.reshape(-1)[:k * b], n - 1)
    g = _sc_gather(x, idx_flat, k * b, d)

    topk = pl.pallas_call(
        functools.partial(_epilogue_kernel, b=b, k=k, d=d),
        grid=(1,),
        in_specs=[
            pl.BlockSpec((b, 1), lambda i: (0, 0)),
            pl.BlockSpec((k * b, d), lambda i: (0, 0)),
        ],
        out_specs=pl.BlockSpec((b, k * d), lambda i: (0, 0)),
        out_shape=jax.ShapeDtypeStruct((b, k * d), jnp.float32),
    )(cnt, g)

    return jnp.concatenate([mean, ssum, topk], axis=-1)
